# Initial kernel scaffold; baseline (speedup 1.0000x reference)
import jax
import jax.numpy as jnp
from jax.experimental import pallas as pl


def kernel(x, edge_index, W_self, W_rel, bias):
    raise NotImplementedError("write your pallas kernel here")


# speedup vs baseline: 5.4998x; 5.4998x over previous
import functools

import jax
import jax.numpy as jnp
from jax import lax
from jax.experimental import pallas as pl
from jax.experimental.pallas import tpu as pltpu
from jax.experimental.pallas import tpu_sc as plsc

CHUNK = 128
NUM_CORES = 2
NUM_SUBCORES = 16
NW = NUM_CORES * NUM_SUBCORES


def _sc_segment_sum(x, src3, dst3, zero, acc_rows, n_chunks):
    n_nodes, feat = x.shape
    rpt = acc_rows // NUM_SUBCORES

    mesh = plsc.VectorSubcoreMesh(core_axis_name="c", subcore_axis_name="s")

    @functools.partial(
        pl.kernel,
        mesh=mesh,
        out_type=jax.ShapeDtypeStruct((NUM_CORES, acc_rows, feat), jnp.float32),
        scratch_types=[
            pltpu.VMEM((n_chunks, CHUNK), jnp.int32),
            pltpu.VMEM((n_chunks, CHUNK), jnp.int32),
            pltpu.VMEM((CHUNK, feat), jnp.float32),
            pltpu.VMEM_SHARED((acc_rows, feat), jnp.float32),
            pltpu.SemaphoreType.DMA,
        ],
    )
    def seg_sum(x_hbm, src_hbm, dst_hbm, zero_hbm, out_hbm,
                src_v, dst_v, rows_v, acc_sh, sem):
        c = lax.axis_index("c")
        s = lax.axis_index("s")
        wid = c * NUM_SUBCORES + s
        pltpu.sync_copy(zero_hbm.at[pl.ds(s * rpt, rpt)],
                        acc_sh.at[pl.ds(s * rpt, rpt)])
        pltpu.sync_copy(src_hbm.at[wid], src_v)
        pltpu.sync_copy(dst_hbm.at[wid], dst_v)
        plsc.subcore_barrier()

        def chunk_body(j, carry):
            pltpu.async_copy(x_hbm.at[src_v.at[j]], rows_v, sem).wait()
            pltpu.sync_copy(rows_v, acc_sh.at[dst_v.at[j]], add=True)
            return carry

        lax.fori_loop(0, n_chunks, chunk_body, 0)
        plsc.subcore_barrier()
        pltpu.sync_copy(acc_sh.at[pl.ds(s * rpt, rpt)],
                        out_hbm.at[c, pl.ds(s * rpt, rpt)])

    return seg_sum(x, src3, dst3, zero)


def _tc_finish(p0, p1, x, w_rel, w_self, bias2d, blk):
    n_nodes, feat = x.shape

    def body(p0_ref, p1_ref, x_ref, wr_ref, ws_ref, b_ref, o_ref):
        agg = p0_ref[...] + p1_ref[...]
        h = jnp.dot(agg, wr_ref[...], preferred_element_type=jnp.float32)
        h = h + jnp.dot(x_ref[...], ws_ref[...], preferred_element_type=jnp.float32)
        o_ref[...] = jnp.maximum(h + b_ref[...], 0.0)

    grid = (n_nodes // blk,)
    row_spec = pl.BlockSpec((blk, feat), lambda i: (i, 0))
    full_spec = pl.BlockSpec((feat, feat), lambda i: (0, 0))
    bias_spec = pl.BlockSpec((1, feat), lambda i: (0, 0))
    return pl.pallas_call(
        body,
        grid=grid,
        in_specs=[row_spec, row_spec, row_spec, full_spec, full_spec, bias_spec],
        out_specs=row_spec,
        out_shape=jax.ShapeDtypeStruct((n_nodes, feat), jnp.float32),
    )(p0, p1, x, w_rel, w_self, bias2d)


def kernel(x, edge_index, W_self, W_rel, bias):
    n_nodes, feat = x.shape
    n_edges = edge_index.shape[1]

    per_w = -(-n_edges // NW)
    n_chunks = -(-per_w // CHUNK)
    padded = NW * n_chunks * CHUNK
    src = edge_index[0].astype(jnp.int32)
    dst = edge_index[1].astype(jnp.int32)
    src3 = jnp.pad(src, (0, padded - n_edges)).reshape(NW, n_chunks, CHUNK)
    dst3 = jnp.pad(dst, (0, padded - n_edges),
                   constant_values=n_nodes).reshape(NW, n_chunks, CHUNK)

    acc_rows = -(-(n_nodes + 1) // (NUM_SUBCORES * 8)) * (NUM_SUBCORES * 8)
    zero = jnp.zeros((acc_rows, feat), jnp.float32)

    partials = _sc_segment_sum(x, src3, dst3, zero, acc_rows, n_chunks)
    p0 = partials[0, :n_nodes]
    p1 = partials[1, :n_nodes]

    blk = 1000
    bias2d = bias.reshape(1, feat)
    return _tc_finish(p0, p1, x, W_rel, W_self, bias2d, blk)
